# baseline (speedup 1.0000x reference)
import functools

import jax
import jax.numpy as jnp
from jax import lax
from jax.experimental import pallas as pl
from jax.experimental.pallas import tpu as pltpu
from jax.experimental.pallas import tpu_sc as plsc

T = 2048
D = 1024
E = 8
FF = 4096
MBLK = 512
NPAD = T * 2 + E * MBLK
NTILES = NPAD // MBLK
NIDX = 2 * T

_SC_NC = 2
_SC_NS = 16
_SC_NW = _SC_NC * _SC_NS
_SC_PER_W = NIDX // _SC_NW
_SC_CH = 64


def _router_body(x_ref, wg_ref, logits_ref, pos1_ref, pos2_ref, w1n_ref,
                 w2n_ref, te_ref, nv_ref):
    x = x_ref[...]
    wg = wg_ref[...]
    logits = lax.dot_general(x, wg, (((1,), (1,)), ((), ())),
                             preferred_element_type=jnp.float32)
    logits_ref[...] = logits

    m = jnp.max(logits, axis=1, keepdims=True)
    p = jnp.exp(logits - m)
    probs = p / jnp.sum(p, axis=1, keepdims=True)

    iota_e = lax.broadcasted_iota(jnp.int32, (T, E), 1)
    a1 = jnp.argmax(probs, axis=1, keepdims=True).astype(jnp.int32)
    m1 = jnp.max(probs, axis=1, keepdims=True)
    eq1 = iota_e == a1
    probs2 = jnp.where(eq1, -1.0, probs)
    a2 = jnp.argmax(probs2, axis=1, keepdims=True).astype(jnp.int32)
    m2 = jnp.max(probs2, axis=1, keepdims=True)
    eq2 = iota_e == a2

    wsum = m1 + m2
    w1n_ref[...] = m1 / wsum
    w2n_ref[...] = m2 / wsum

    ohsum = (eq1 | eq2).astype(jnp.bfloat16)
    ti = lax.broadcasted_iota(jnp.int32, (T, T), 0)
    tj = lax.broadcasted_iota(jnp.int32, (T, T), 1)
    tril = (tj < ti).astype(jnp.bfloat16)
    cum = lax.dot_general(tril, ohsum, (((1,), (0,)), ((), ())),
                          preferred_element_type=jnp.float32)
    cum_i = cum.astype(jnp.int32)

    cnt = jnp.sum(eq1.astype(jnp.int32) + eq2.astype(jnp.int32), axis=0,
                  keepdims=True)
    pc = ((cnt + (MBLK - 1)) // MBLK) * MBLK
    lane = lax.broadcasted_iota(jnp.int32, (1, E), 1)
    off = jnp.zeros((1, E), jnp.int32)
    for k in range(1, E):
        off = off + jnp.where(lane >= k, jnp.roll(pc, k, axis=1), 0)

    rank1 = jnp.sum(jnp.where(eq1, cum_i, 0), axis=1, keepdims=True)
    rank2 = jnp.sum(jnp.where(eq2, cum_i, 0), axis=1, keepdims=True)
    off1 = jnp.sum(jnp.where(eq1, off, 0), axis=1, keepdims=True)
    off2 = jnp.sum(jnp.where(eq2, off, 0), axis=1, keepdims=True)
    pos1_ref[...] = off1 + rank1
    pos2_ref[...] = off2 + rank2

    bend = (off + pc) // MBLK
    iota_t = lax.broadcasted_iota(jnp.int32, (NTILES, E), 0)
    te = jnp.sum((iota_t >= bend).astype(jnp.int32), axis=1, keepdims=True)
    te_ref[...] = jnp.minimum(te, E - 1)
    nv_ref[...] = jnp.sum(pc, axis=1, keepdims=True) // MBLK


def _router(x, Wg):
    return pl.pallas_call(
        _router_body,
        out_shape=[
            jax.ShapeDtypeStruct((T, E), jnp.float32),
            jax.ShapeDtypeStruct((T, 1), jnp.int32),
            jax.ShapeDtypeStruct((T, 1), jnp.int32),
            jax.ShapeDtypeStruct((T, 1), jnp.float32),
            jax.ShapeDtypeStruct((T, 1), jnp.float32),
            jax.ShapeDtypeStruct((NTILES, 1), jnp.int32),
            jax.ShapeDtypeStruct((1, 1), jnp.int32),
        ],
    )(x, Wg)


def _sc_mesh():
    return plsc.VectorSubcoreMesh(core_axis_name="c", subcore_axis_name="s")


def _sc_dispatch(x2d, pos_cat):
    @functools.partial(
        pl.kernel, mesh=_sc_mesh(),
        out_type=jax.ShapeDtypeStruct((NPAD, D), jnp.float32),
        scratch_types=[pltpu.VMEM((_SC_CH,), jnp.int32),
                       pltpu.VMEM((_SC_CH, D), jnp.float32),
                       pltpu.SemaphoreType.DMA])
    def k(x_hbm, idx_hbm, out_hbm, idx_v, rows_v, sem):
        wid = lax.axis_index("s") * _SC_NC + lax.axis_index("c")
        base = wid * _SC_PER_W
        for c in range(_SC_PER_W // _SC_CH):
            b0 = base + c * _SC_CH
            tok = lax.rem(b0, T)
            pltpu.sync_copy(idx_hbm.at[pl.ds(b0, _SC_CH)], idx_v)
            pltpu.sync_copy(x_hbm.at[pl.ds(tok, _SC_CH)], rows_v)
            pltpu.async_copy(rows_v, out_hbm.at[idx_v], sem).wait()

    return k(x2d, pos_cat)


def _sc_combine(ys, pos_cat):
    @functools.partial(
        pl.kernel, mesh=_sc_mesh(),
        out_type=jax.ShapeDtypeStruct((NIDX, D), jnp.float32),
        scratch_types=[pltpu.VMEM((_SC_CH,), jnp.int32),
                       pltpu.VMEM((_SC_CH, D), jnp.float32),
                       pltpu.SemaphoreType.DMA])
    def k(ys_hbm, idx_hbm, out_hbm, idx_v, rows_v, sem):
        wid = lax.axis_index("s") * _SC_NC + lax.axis_index("c")
        base = wid * _SC_PER_W
        for c in range(_SC_PER_W // _SC_CH):
            b0 = base + c * _SC_CH
            pltpu.sync_copy(idx_hbm.at[pl.ds(b0, _SC_CH)], idx_v)
            pltpu.async_copy(ys_hbm.at[idx_v], rows_v, sem).wait()
            pltpu.sync_copy(rows_v, out_hbm.at[pl.ds(b0, _SC_CH)])

    return k(ys, pos_cat)


def _gelu_exact(x):
    return 0.5 * x * (1.0 + lax.erf(x * 0.7071067811865476))


NFSPLIT = 4
FCHUNK = FF // NFSPLIT


def _mlp_fused_body(te_ref, nv_ref, xs_ref, w1_ref, w2_ref, ys_ref):
    i = pl.program_id(0)
    f = pl.program_id(1)

    @pl.when(i < nv_ref[0])
    def _():
        xb = xs_ref[...].astype(jnp.bfloat16)
        w1 = w1_ref[0].astype(jnp.bfloat16)
        acc = lax.dot_general(xb, w1, (((1,), (1,)), ((), ())),
                              preferred_element_type=jnp.float32)
        hb = _gelu_exact(acc).astype(jnp.bfloat16)
        w2 = w2_ref[0].astype(jnp.bfloat16)
        part = lax.dot_general(hb, w2, (((1,), (1,)), ((), ())),
                               preferred_element_type=jnp.float32)

        @pl.when(f == 0)
        def _():
            ys_ref[...] = part

        @pl.when(f != 0)
        def _():
            ys_ref[...] += part


def _snake(i, f):
    return jnp.where(lax.rem(i, 2) == 0, f, NFSPLIT - 1 - f)


def _mlp_fused(xs, W1, W2, te, nv):
    grid_spec = pltpu.PrefetchScalarGridSpec(
        num_scalar_prefetch=2,
        grid=(NTILES, NFSPLIT),
        in_specs=[
            pl.BlockSpec((MBLK, D), lambda i, f, te, nv: (i, 0)),
            pl.BlockSpec((1, FCHUNK, D),
                         lambda i, f, te, nv: (te[i], _snake(i, f), 0)),
            pl.BlockSpec((1, D, FCHUNK),
                         lambda i, f, te, nv: (te[i], 0, _snake(i, f))),
        ],
        out_specs=pl.BlockSpec((MBLK, D), lambda i, f, te, nv: (i, 0)),
    )
    return pl.pallas_call(
        _mlp_fused_body,
        grid_spec=grid_spec,
        out_shape=jax.ShapeDtypeStruct((NPAD, D), jnp.float32),
        compiler_params=pltpu.CompilerParams(
            dimension_semantics=("parallel", "arbitrary")),
    )(te, nv, xs, W1, W2)


def _mlp1_body(te_ref, nv_ref, xs_ref, w1_ref, h_ref):
    i = pl.program_id(0)

    @pl.when(i < nv_ref[0])
    def _():
        xb = xs_ref[...].astype(jnp.bfloat16)
        w = w1_ref[0].astype(jnp.bfloat16)
        acc = lax.dot_general(xb, w, (((1,), (1,)), ((), ())),
                              preferred_element_type=jnp.float32)
        h_ref[...] = _gelu_exact(acc).astype(jnp.bfloat16)


def _mlp1(xs, W1, te, nv):
    grid_spec = pltpu.PrefetchScalarGridSpec(
        num_scalar_prefetch=2,
        grid=(NTILES,),
        in_specs=[
            pl.BlockSpec((MBLK, D), lambda i, te, nv: (i, 0)),
            pl.BlockSpec((1, FF, D), lambda i, te, nv: (te[i], 0, 0)),
        ],
        out_specs=pl.BlockSpec((MBLK, FF), lambda i, te, nv: (i, 0)),
    )
    return pl.pallas_call(
        _mlp1_body,
        grid_spec=grid_spec,
        out_shape=jax.ShapeDtypeStruct((NPAD, FF), jnp.bfloat16),
        compiler_params=pltpu.CompilerParams(
            dimension_semantics=("parallel",)),
    )(te, nv, xs, W1)


def _mlp2_body(te_ref, nv_ref, h_ref, w2_ref, ys_ref):
    i = pl.program_id(0)

    @pl.when(i < nv_ref[0])
    def _():
        w = w2_ref[0].astype(jnp.bfloat16)
        ys_ref[...] = lax.dot_general(h_ref[...], w, (((1,), (1,)), ((), ())),
                                      preferred_element_type=jnp.float32)


def _mlp2(h, W2, te, nv):
    grid_spec = pltpu.PrefetchScalarGridSpec(
        num_scalar_prefetch=2,
        grid=(NTILES,),
        in_specs=[
            pl.BlockSpec((MBLK, FF), lambda i, te, nv: (i, 0)),
            pl.BlockSpec((1, D, FF), lambda i, te, nv: (te[i], 0, 0)),
        ],
        out_specs=pl.BlockSpec((MBLK, D), lambda i, te, nv: (i, 0)),
    )
    return pl.pallas_call(
        _mlp2_body,
        grid_spec=grid_spec,
        out_shape=jax.ShapeDtypeStruct((NPAD, D), jnp.float32),
        compiler_params=pltpu.CompilerParams(
            dimension_semantics=("parallel",)),
    )(te, nv, h, W2)


def _combine_body(g1_ref, g2_ref, w1_ref, w2_ref, o_ref):
    o_ref[...] = g1_ref[...] * w1_ref[...] + g2_ref[...] * w2_ref[...]


def _combine(g, w1n, w2n):
    nblk = T // MBLK
    return pl.pallas_call(
        _combine_body,
        grid=(nblk,),
        in_specs=[
            pl.BlockSpec((MBLK, D), lambda i: (i, 0)),
            pl.BlockSpec((MBLK, D), lambda i: (i + nblk, 0)),
            pl.BlockSpec((MBLK, 1), lambda i: (i, 0)),
            pl.BlockSpec((MBLK, 1), lambda i: (i, 0)),
        ],
        out_specs=pl.BlockSpec((MBLK, D), lambda i: (i, 0)),
        out_shape=jax.ShapeDtypeStruct((T, D), jnp.float32),
    )(g, g, w1n, w2n)


def kernel(hidden_states, Wg, W1, W2):
    b, s, d = hidden_states.shape
    x = hidden_states.reshape(T, D)

    logits, pos1, pos2, w1n, w2n, te, nv = _router(x, Wg)
    pos_cat = jnp.concatenate([pos1[:, 0], pos2[:, 0]])

    xs = _sc_dispatch(x, pos_cat)
    ys = _mlp_fused(xs, W1, W2, te[:, 0], nv[0])
    g = _sc_combine(ys, pos_cat)
    out = _combine(g, w1n, w2n)

    return (out.reshape(b, s, d), logits)


# speedup vs baseline: 1.0991x; 1.0991x over previous
import functools

import jax
import jax.numpy as jnp
from jax import lax
from jax.experimental import pallas as pl
from jax.experimental.pallas import tpu as pltpu
from jax.experimental.pallas import tpu_sc as plsc

T = 2048
D = 1024
E = 8
FF = 4096
MBLK = 512
NPAD = T * 2 + E * MBLK
NTILES = NPAD // MBLK
NIDX = 2 * T

_SC_NC = 2
_SC_NS = 16
_SC_NW = _SC_NC * _SC_NS
_SC_PER_W = NIDX // _SC_NW
_SC_CH = 32
_SC_NCH = _SC_PER_W // _SC_CH


def _router_body(x_ref, wg_ref, logits_ref, pos1_ref, pos2_ref, w1n_ref,
                 w2n_ref, te_ref, nv_ref):
    x = x_ref[...]
    wg = wg_ref[...]
    logits = lax.dot_general(x, wg, (((1,), (1,)), ((), ())),
                             preferred_element_type=jnp.float32)
    logits_ref[...] = logits

    m = jnp.max(logits, axis=1, keepdims=True)
    p = jnp.exp(logits - m)
    probs = p / jnp.sum(p, axis=1, keepdims=True)

    iota_e = lax.broadcasted_iota(jnp.int32, (T, E), 1)
    a1 = jnp.argmax(probs, axis=1, keepdims=True).astype(jnp.int32)
    m1 = jnp.max(probs, axis=1, keepdims=True)
    eq1 = iota_e == a1
    probs2 = jnp.where(eq1, -1.0, probs)
    a2 = jnp.argmax(probs2, axis=1, keepdims=True).astype(jnp.int32)
    m2 = jnp.max(probs2, axis=1, keepdims=True)
    eq2 = iota_e == a2

    wsum = m1 + m2
    w1n_ref[...] = m1 / wsum
    w2n_ref[...] = m2 / wsum

    ohsum = (eq1 | eq2).astype(jnp.bfloat16)
    ti = lax.broadcasted_iota(jnp.int32, (T, T), 0)
    tj = lax.broadcasted_iota(jnp.int32, (T, T), 1)
    tril = (tj < ti).astype(jnp.bfloat16)
    cum = lax.dot_general(tril, ohsum, (((1,), (0,)), ((), ())),
                          preferred_element_type=jnp.float32)
    cum_i = cum.astype(jnp.int32)

    cnt = jnp.sum(eq1.astype(jnp.int32) + eq2.astype(jnp.int32), axis=0,
                  keepdims=True)
    pc = ((cnt + (MBLK - 1)) // MBLK) * MBLK
    lane = lax.broadcasted_iota(jnp.int32, (1, E), 1)
    off = jnp.zeros((1, E), jnp.int32)
    for k in range(1, E):
        off = off + jnp.where(lane >= k, jnp.roll(pc, k, axis=1), 0)

    rank1 = jnp.sum(jnp.where(eq1, cum_i, 0), axis=1, keepdims=True)
    rank2 = jnp.sum(jnp.where(eq2, cum_i, 0), axis=1, keepdims=True)
    off1 = jnp.sum(jnp.where(eq1, off, 0), axis=1, keepdims=True)
    off2 = jnp.sum(jnp.where(eq2, off, 0), axis=1, keepdims=True)
    pos1_ref[...] = off1 + rank1
    pos2_ref[...] = off2 + rank2

    bend = (off + pc) // MBLK
    iota_t = lax.broadcasted_iota(jnp.int32, (NTILES, E), 0)
    te = jnp.sum((iota_t >= bend).astype(jnp.int32), axis=1, keepdims=True)
    te_ref[...] = jnp.minimum(te, E - 1)
    nv_ref[...] = jnp.sum(pc, axis=1, keepdims=True) // MBLK


def _router(x, Wg):
    return pl.pallas_call(
        _router_body,
        out_shape=[
            jax.ShapeDtypeStruct((T, E), jnp.float32),
            jax.ShapeDtypeStruct((T, 1), jnp.int32),
            jax.ShapeDtypeStruct((T, 1), jnp.int32),
            jax.ShapeDtypeStruct((T, 1), jnp.float32),
            jax.ShapeDtypeStruct((T, 1), jnp.float32),
            jax.ShapeDtypeStruct((NTILES, 1), jnp.int32),
            jax.ShapeDtypeStruct((1, 1), jnp.int32),
        ],
    )(x, Wg)


def _sc_mesh():
    return plsc.VectorSubcoreMesh(core_axis_name="c", subcore_axis_name="s")


def _sc_dispatch(x2d, pos_cat):
    @functools.partial(
        pl.kernel, mesh=_sc_mesh(),
        out_type=jax.ShapeDtypeStruct((NPAD, D), jnp.float32),
        scratch_types=[pltpu.VMEM((2, _SC_CH), jnp.int32),
                       pltpu.VMEM((2, _SC_CH, D), jnp.float32),
                       pltpu.SemaphoreType.DMA((2,))])
    def k(x_hbm, idx_hbm, out_hbm, idx_v, rows_v, sem):
        wid = lax.axis_index("s") * _SC_NC + lax.axis_index("c")
        base = wid * _SC_PER_W
        scatters = [None, None]
        for c in range(_SC_NCH):
            b = c % 2
            b0 = base + c * _SC_CH
            tok = lax.rem(b0, T)
            if scatters[b] is not None:
                scatters[b].wait()
            pltpu.sync_copy(idx_hbm.at[pl.ds(b0, _SC_CH)], idx_v.at[b])
            pltpu.sync_copy(x_hbm.at[pl.ds(tok, _SC_CH)], rows_v.at[b])
            scatters[b] = pltpu.async_copy(rows_v.at[b],
                                           out_hbm.at[idx_v.at[b]],
                                           sem.at[b])
        for cp in scatters:
            cp.wait()

    return k(x2d, pos_cat)


def _sc_combine(ys, pos_cat):
    @functools.partial(
        pl.kernel, mesh=_sc_mesh(),
        out_type=jax.ShapeDtypeStruct((NIDX, D), jnp.float32),
        scratch_types=[pltpu.VMEM((2, _SC_CH), jnp.int32),
                       pltpu.VMEM((2, _SC_CH, D), jnp.float32),
                       pltpu.SemaphoreType.DMA((2,)),
                       pltpu.SemaphoreType.DMA((2,))])
    def k(ys_hbm, idx_hbm, out_hbm, idx_v, rows_v, gsem, ssem):
        wid = lax.axis_index("s") * _SC_NC + lax.axis_index("c")
        base = wid * _SC_PER_W
        gathers = [None, None]
        stores = [None, None]
        for c in range(_SC_NCH + 1):
            b = c % 2
            if c < _SC_NCH:
                b0 = base + c * _SC_CH
                if stores[b] is not None:
                    stores[b].wait()
                pltpu.sync_copy(idx_hbm.at[pl.ds(b0, _SC_CH)], idx_v.at[b])
                gathers[b] = pltpu.async_copy(ys_hbm.at[idx_v.at[b]],
                                              rows_v.at[b], gsem.at[b])
            if c > 0:
                bp = (c - 1) % 2
                bp0 = base + (c - 1) * _SC_CH
                gathers[bp].wait()
                stores[bp] = pltpu.async_copy(rows_v.at[bp],
                                              out_hbm.at[pl.ds(bp0, _SC_CH)],
                                              ssem.at[bp])
        for cp in stores:
            cp.wait()

    return k(ys, pos_cat)


def _gelu_exact(x):
    return 0.5 * x * (1.0 + lax.erf(x * 0.7071067811865476))


NFSPLIT = 2
FCHUNK = FF // NFSPLIT


def _mlp_fused_body(te_ref, nv_ref, xs_ref, w1_ref, w2_ref, ys_ref):
    i = pl.program_id(0)
    f = pl.program_id(1)

    @pl.when(i < nv_ref[0])
    def _():
        xb = xs_ref[...].astype(jnp.bfloat16)
        w1 = w1_ref[0].astype(jnp.bfloat16)
        acc = lax.dot_general(xb, w1, (((1,), (1,)), ((), ())),
                              preferred_element_type=jnp.float32)
        hb = _gelu_exact(acc).astype(jnp.bfloat16)
        w2 = w2_ref[0].astype(jnp.bfloat16)
        part = lax.dot_general(hb, w2, (((1,), (1,)), ((), ())),
                               preferred_element_type=jnp.float32)

        @pl.when(f == 0)
        def _():
            ys_ref[...] = part

        @pl.when(f != 0)
        def _():
            ys_ref[...] += part


def _snake(i, f):
    return jnp.where(lax.rem(i, 2) == 0, f, NFSPLIT - 1 - f)


def _mlp_fused(xs, W1, W2, te, nv):
    grid_spec = pltpu.PrefetchScalarGridSpec(
        num_scalar_prefetch=2,
        grid=(NTILES, NFSPLIT),
        in_specs=[
            pl.BlockSpec((MBLK, D), lambda i, f, te, nv: (i, 0)),
            pl.BlockSpec((1, FCHUNK, D),
                         lambda i, f, te, nv: (te[i], _snake(i, f), 0)),
            pl.BlockSpec((1, D, FCHUNK),
                         lambda i, f, te, nv: (te[i], 0, _snake(i, f))),
        ],
        out_specs=pl.BlockSpec((MBLK, D), lambda i, f, te, nv: (i, 0)),
    )
    return pl.pallas_call(
        _mlp_fused_body,
        grid_spec=grid_spec,
        out_shape=jax.ShapeDtypeStruct((NPAD, D), jnp.float32),
        compiler_params=pltpu.CompilerParams(
            dimension_semantics=("parallel", "arbitrary")),
    )(te, nv, xs, W1, W2)


def _mlp1_body(te_ref, nv_ref, xs_ref, w1_ref, h_ref):
    i = pl.program_id(0)

    @pl.when(i < nv_ref[0])
    def _():
        xb = xs_ref[...].astype(jnp.bfloat16)
        w = w1_ref[0].astype(jnp.bfloat16)
        acc = lax.dot_general(xb, w, (((1,), (1,)), ((), ())),
                              preferred_element_type=jnp.float32)
        h_ref[...] = _gelu_exact(acc).astype(jnp.bfloat16)


def _mlp1(xs, W1, te, nv):
    grid_spec = pltpu.PrefetchScalarGridSpec(
        num_scalar_prefetch=2,
        grid=(NTILES,),
        in_specs=[
            pl.BlockSpec((MBLK, D), lambda i, te, nv: (i, 0)),
            pl.BlockSpec((1, FF, D), lambda i, te, nv: (te[i], 0, 0)),
        ],
        out_specs=pl.BlockSpec((MBLK, FF), lambda i, te, nv: (i, 0)),
    )
    return pl.pallas_call(
        _mlp1_body,
        grid_spec=grid_spec,
        out_shape=jax.ShapeDtypeStruct((NPAD, FF), jnp.bfloat16),
        compiler_params=pltpu.CompilerParams(
            dimension_semantics=("parallel",)),
    )(te, nv, xs, W1)


def _mlp2_body(te_ref, nv_ref, h_ref, w2_ref, ys_ref):
    i = pl.program_id(0)

    @pl.when(i < nv_ref[0])
    def _():
        w = w2_ref[0].astype(jnp.bfloat16)
        ys_ref[...] = lax.dot_general(h_ref[...], w, (((1,), (1,)), ((), ())),
                                      preferred_element_type=jnp.float32)


def _mlp2(h, W2, te, nv):
    grid_spec = pltpu.PrefetchScalarGridSpec(
        num_scalar_prefetch=2,
        grid=(NTILES,),
        in_specs=[
            pl.BlockSpec((MBLK, FF), lambda i, te, nv: (i, 0)),
            pl.BlockSpec((1, D, FF), lambda i, te, nv: (te[i], 0, 0)),
        ],
        out_specs=pl.BlockSpec((MBLK, D), lambda i, te, nv: (i, 0)),
    )
    return pl.pallas_call(
        _mlp2_body,
        grid_spec=grid_spec,
        out_shape=jax.ShapeDtypeStruct((NPAD, D), jnp.float32),
        compiler_params=pltpu.CompilerParams(
            dimension_semantics=("parallel",)),
    )(te, nv, h, W2)


def _combine_body(g1_ref, g2_ref, w1_ref, w2_ref, o_ref):
    o_ref[...] = g1_ref[...] * w1_ref[...] + g2_ref[...] * w2_ref[...]


def _combine(g, w1n, w2n):
    nblk = T // MBLK
    return pl.pallas_call(
        _combine_body,
        grid=(nblk,),
        in_specs=[
            pl.BlockSpec((MBLK, D), lambda i: (i, 0)),
            pl.BlockSpec((MBLK, D), lambda i: (i + nblk, 0)),
            pl.BlockSpec((MBLK, 1), lambda i: (i, 0)),
            pl.BlockSpec((MBLK, 1), lambda i: (i, 0)),
        ],
        out_specs=pl.BlockSpec((MBLK, D), lambda i: (i, 0)),
        out_shape=jax.ShapeDtypeStruct((T, D), jnp.float32),
    )(g, g, w1n, w2n)


def kernel(hidden_states, Wg, W1, W2):
    b, s, d = hidden_states.shape
    x = hidden_states.reshape(T, D)

    logits, pos1, pos2, w1n, w2n, te, nv = _router(x, Wg)
    pos_cat = jnp.concatenate([pos1[:, 0], pos2[:, 0]])

    xs = _sc_dispatch(x, pos_cat)
    ys = _mlp_fused(xs, W1, W2, te[:, 0], nv[0])
    g = _sc_combine(ys, pos_cat)
    out = _combine(g, w1n, w2n)

    return (out.reshape(b, s, d), logits)


# speedup vs baseline: 1.1210x; 1.0200x over previous
import functools

import jax
import jax.numpy as jnp
from jax import lax
from jax.experimental import pallas as pl
from jax.experimental.pallas import tpu as pltpu
from jax.experimental.pallas import tpu_sc as plsc

T = 2048
D = 1024
E = 8
FF = 4096
MBLK = 512
NPAD = T * 2 + E * MBLK
NTILES = NPAD // MBLK
NIDX = 2 * T

_SC_NC = 2
_SC_NS = 16
_SC_NW = _SC_NC * _SC_NS
_SC_PER_W = NIDX // _SC_NW
_SC_CH = 64


def _router_body(x_ref, wg_ref, logits_ref, pos1_ref, pos2_ref, w1n_ref,
                 w2n_ref, te_ref, nv_ref):
    x = x_ref[...]
    wg = wg_ref[...]
    logits = lax.dot_general(x, wg, (((1,), (1,)), ((), ())),
                             preferred_element_type=jnp.float32)
    logits_ref[...] = logits

    m = jnp.max(logits, axis=1, keepdims=True)
    p = jnp.exp(logits - m)
    probs = p / jnp.sum(p, axis=1, keepdims=True)

    iota_e = lax.broadcasted_iota(jnp.int32, (T, E), 1)
    a1 = jnp.argmax(probs, axis=1, keepdims=True).astype(jnp.int32)
    m1 = jnp.max(probs, axis=1, keepdims=True)
    eq1 = iota_e == a1
    probs2 = jnp.where(eq1, -1.0, probs)
    a2 = jnp.argmax(probs2, axis=1, keepdims=True).astype(jnp.int32)
    m2 = jnp.max(probs2, axis=1, keepdims=True)
    eq2 = iota_e == a2

    wsum = m1 + m2
    w1n_ref[...] = m1 / wsum
    w2n_ref[...] = m2 / wsum

    ohsum = (eq1 | eq2).astype(jnp.bfloat16)
    ti = lax.broadcasted_iota(jnp.int32, (T, T), 0)
    tj = lax.broadcasted_iota(jnp.int32, (T, T), 1)
    tril = (tj < ti).astype(jnp.bfloat16)
    cum = lax.dot_general(tril, ohsum, (((1,), (0,)), ((), ())),
                          preferred_element_type=jnp.float32)
    cum_i = cum.astype(jnp.int32)

    cnt = jnp.sum(eq1.astype(jnp.int32) + eq2.astype(jnp.int32), axis=0,
                  keepdims=True)
    pc = ((cnt + (MBLK - 1)) // MBLK) * MBLK
    lane = lax.broadcasted_iota(jnp.int32, (1, E), 1)
    off = jnp.zeros((1, E), jnp.int32)
    for k in range(1, E):
        off = off + jnp.where(lane >= k, jnp.roll(pc, k, axis=1), 0)

    rank1 = jnp.sum(jnp.where(eq1, cum_i, 0), axis=1, keepdims=True)
    rank2 = jnp.sum(jnp.where(eq2, cum_i, 0), axis=1, keepdims=True)
    off1 = jnp.sum(jnp.where(eq1, off, 0), axis=1, keepdims=True)
    off2 = jnp.sum(jnp.where(eq2, off, 0), axis=1, keepdims=True)
    pos1_ref[...] = off1 + rank1
    pos2_ref[...] = off2 + rank2

    bend = (off + pc) // MBLK
    iota_t = lax.broadcasted_iota(jnp.int32, (NTILES, E), 0)
    te = jnp.sum((iota_t >= bend).astype(jnp.int32), axis=1, keepdims=True)
    te_ref[...] = jnp.minimum(te, E - 1)
    nv_ref[...] = jnp.sum(pc, axis=1, keepdims=True) // MBLK


def _router(x, Wg):
    return pl.pallas_call(
        _router_body,
        out_shape=[
            jax.ShapeDtypeStruct((T, E), jnp.float32),
            jax.ShapeDtypeStruct((T, 1), jnp.int32),
            jax.ShapeDtypeStruct((T, 1), jnp.int32),
            jax.ShapeDtypeStruct((T, 1), jnp.float32),
            jax.ShapeDtypeStruct((T, 1), jnp.float32),
            jax.ShapeDtypeStruct((NTILES, 1), jnp.int32),
            jax.ShapeDtypeStruct((1, 1), jnp.int32),
        ],
    )(x, Wg)


def _sc_mesh():
    return plsc.VectorSubcoreMesh(core_axis_name="c", subcore_axis_name="s")


def _sc_dispatch(x2d, pos_cat):
    @functools.partial(
        pl.kernel, mesh=_sc_mesh(),
        out_type=jax.ShapeDtypeStruct((NPAD, D), jnp.float32),
        scratch_types=[pltpu.VMEM((_SC_CH,), jnp.int32),
                       pltpu.VMEM((_SC_CH, D), jnp.float32),
                       pltpu.SemaphoreType.DMA])
    def k(x_hbm, idx_hbm, out_hbm, idx_v, rows_v, sem):
        wid = lax.axis_index("s") * _SC_NC + lax.axis_index("c")
        base = wid * _SC_PER_W
        for c in range(_SC_PER_W // _SC_CH):
            b0 = base + c * _SC_CH
            tok = lax.rem(b0, T)
            pltpu.sync_copy(idx_hbm.at[pl.ds(b0, _SC_CH)], idx_v)
            pltpu.sync_copy(x_hbm.at[pl.ds(tok, _SC_CH)], rows_v)
            pltpu.async_copy(rows_v, out_hbm.at[idx_v], sem).wait()

    return k(x2d, pos_cat)


def _sc_combine(ys, pos_cat):
    @functools.partial(
        pl.kernel, mesh=_sc_mesh(),
        out_type=jax.ShapeDtypeStruct((NIDX, D), jnp.float32),
        scratch_types=[pltpu.VMEM((_SC_CH,), jnp.int32),
                       pltpu.VMEM((_SC_CH, D), jnp.float32),
                       pltpu.SemaphoreType.DMA])
    def k(ys_hbm, idx_hbm, out_hbm, idx_v, rows_v, sem):
        wid = lax.axis_index("s") * _SC_NC + lax.axis_index("c")
        base = wid * _SC_PER_W
        for c in range(_SC_PER_W // _SC_CH):
            b0 = base + c * _SC_CH
            pltpu.sync_copy(idx_hbm.at[pl.ds(b0, _SC_CH)], idx_v)
            pltpu.async_copy(ys_hbm.at[idx_v], rows_v, sem).wait()
            pltpu.sync_copy(rows_v, out_hbm.at[pl.ds(b0, _SC_CH)])

    return k(ys, pos_cat)


def _gelu_exact(x):
    return 0.5 * x * (1.0 + lax.erf(x * 0.7071067811865476))


NFSPLIT = 2
FCHUNK = FF // NFSPLIT


def _mlp_fused_body(te_ref, nv_ref, xs_ref, w1_ref, w2_ref, ys_ref):
    i = pl.program_id(0)
    f = pl.program_id(1)

    @pl.when(i < nv_ref[0])
    def _():
        xb = xs_ref[...].astype(jnp.bfloat16)
        w1 = w1_ref[0].astype(jnp.bfloat16)
        acc = lax.dot_general(xb, w1, (((1,), (1,)), ((), ())),
                              preferred_element_type=jnp.float32)
        hb = _gelu_exact(acc).astype(jnp.bfloat16)
        w2 = w2_ref[0].astype(jnp.bfloat16)
        part = lax.dot_general(hb, w2, (((1,), (1,)), ((), ())),
                               preferred_element_type=jnp.float32)

        @pl.when(f == 0)
        def _():
            ys_ref[...] = part

        @pl.when(f != 0)
        def _():
            ys_ref[...] += part


def _snake(i, f):
    return jnp.where(lax.rem(i, 2) == 0, f, NFSPLIT - 1 - f)


def _mlp_fused(xs, W1, W2, te, nv):
    grid_spec = pltpu.PrefetchScalarGridSpec(
        num_scalar_prefetch=2,
        grid=(NTILES, NFSPLIT),
        in_specs=[
            pl.BlockSpec((MBLK, D),
                         lambda i, f, te, nv: (jnp.minimum(i, nv[0]), 0)),
            pl.BlockSpec((1, FCHUNK, D),
                         lambda i, f, te, nv: (te[i], _snake(i, f), 0)),
            pl.BlockSpec((1, D, FCHUNK),
                         lambda i, f, te, nv: (te[i], 0, _snake(i, f))),
        ],
        out_specs=pl.BlockSpec(
            (MBLK, D), lambda i, f, te, nv: (jnp.minimum(i, nv[0]), 0)),
    )
    return pl.pallas_call(
        _mlp_fused_body,
        grid_spec=grid_spec,
        out_shape=jax.ShapeDtypeStruct((NPAD, D), jnp.float32),
        compiler_params=pltpu.CompilerParams(
            dimension_semantics=("parallel", "arbitrary")),
    )(te, nv, xs, W1, W2)


def _mlp1_body(te_ref, nv_ref, xs_ref, w1_ref, h_ref):
    i = pl.program_id(0)

    @pl.when(i < nv_ref[0])
    def _():
        xb = xs_ref[...].astype(jnp.bfloat16)
        w = w1_ref[0].astype(jnp.bfloat16)
        acc = lax.dot_general(xb, w, (((1,), (1,)), ((), ())),
                              preferred_element_type=jnp.float32)
        h_ref[...] = _gelu_exact(acc).astype(jnp.bfloat16)


def _mlp1(xs, W1, te, nv):
    grid_spec = pltpu.PrefetchScalarGridSpec(
        num_scalar_prefetch=2,
        grid=(NTILES,),
        in_specs=[
            pl.BlockSpec((MBLK, D), lambda i, te, nv: (i, 0)),
            pl.BlockSpec((1, FF, D), lambda i, te, nv: (te[i], 0, 0)),
        ],
        out_specs=pl.BlockSpec((MBLK, FF), lambda i, te, nv: (i, 0)),
    )
    return pl.pallas_call(
        _mlp1_body,
        grid_spec=grid_spec,
        out_shape=jax.ShapeDtypeStruct((NPAD, FF), jnp.bfloat16),
        compiler_params=pltpu.CompilerParams(
            dimension_semantics=("parallel",)),
    )(te, nv, xs, W1)


def _mlp2_body(te_ref, nv_ref, h_ref, w2_ref, ys_ref):
    i = pl.program_id(0)

    @pl.when(i < nv_ref[0])
    def _():
        w = w2_ref[0].astype(jnp.bfloat16)
        ys_ref[...] = lax.dot_general(h_ref[...], w, (((1,), (1,)), ((), ())),
                                      preferred_element_type=jnp.float32)


def _mlp2(h, W2, te, nv):
    grid_spec = pltpu.PrefetchScalarGridSpec(
        num_scalar_prefetch=2,
        grid=(NTILES,),
        in_specs=[
            pl.BlockSpec((MBLK, FF), lambda i, te, nv: (i, 0)),
            pl.BlockSpec((1, D, FF), lambda i, te, nv: (te[i], 0, 0)),
        ],
        out_specs=pl.BlockSpec((MBLK, D), lambda i, te, nv: (i, 0)),
    )
    return pl.pallas_call(
        _mlp2_body,
        grid_spec=grid_spec,
        out_shape=jax.ShapeDtypeStruct((NPAD, D), jnp.float32),
        compiler_params=pltpu.CompilerParams(
            dimension_semantics=("parallel",)),
    )(te, nv, h, W2)


def _combine_body(g1_ref, g2_ref, w1_ref, w2_ref, o_ref):
    o_ref[...] = g1_ref[...] * w1_ref[...] + g2_ref[...] * w2_ref[...]


def _combine(g, w1n, w2n):
    nblk = T // MBLK
    return pl.pallas_call(
        _combine_body,
        grid=(nblk,),
        in_specs=[
            pl.BlockSpec((MBLK, D), lambda i: (i, 0)),
            pl.BlockSpec((MBLK, D), lambda i: (i + nblk, 0)),
            pl.BlockSpec((MBLK, 1), lambda i: (i, 0)),
            pl.BlockSpec((MBLK, 1), lambda i: (i, 0)),
        ],
        out_specs=pl.BlockSpec((MBLK, D), lambda i: (i, 0)),
        out_shape=jax.ShapeDtypeStruct((T, D), jnp.float32),
    )(g, g, w1n, w2n)


def kernel(hidden_states, Wg, W1, W2):
    b, s, d = hidden_states.shape
    x = hidden_states.reshape(T, D)

    logits, pos1, pos2, w1n, w2n, te, nv = _router(x, Wg)
    pos_cat = jnp.concatenate([pos1[:, 0], pos2[:, 0]])

    xs = _sc_dispatch(x, pos_cat)
    ys = _mlp_fused(xs, W1, W2, te[:, 0], nv[0])
    g = _sc_combine(ys, pos_cat)
    out = _combine(g, w1n, w2n)

    return (out.reshape(b, s, d), logits)


# speedup vs baseline: 1.1229x; 1.0017x over previous
import functools

import jax
import jax.numpy as jnp
from jax import lax
from jax.experimental import pallas as pl
from jax.experimental.pallas import tpu as pltpu
from jax.experimental.pallas import tpu_sc as plsc

T = 2048
D = 1024
E = 8
FF = 4096
MBLK = 512
NPAD = T * 2 + E * MBLK
NTILES = NPAD // MBLK
NIDX = 2 * T

_SC_NC = 2
_SC_NS = 16
_SC_NW = _SC_NC * _SC_NS
_SC_PER_W = NIDX // _SC_NW
_SC_CH = 64


def _router_body(x_ref, wg_ref, logits_ref, pos1_ref, pos2_ref, w1n_ref,
                 w2n_ref, te_ref, nv_ref):
    x = x_ref[...]
    wg = wg_ref[...]
    logits = lax.dot_general(x, wg, (((1,), (1,)), ((), ())),
                             preferred_element_type=jnp.float32)
    logits_ref[...] = logits

    m = jnp.max(logits, axis=1, keepdims=True)
    p = jnp.exp(logits - m)
    probs = p / jnp.sum(p, axis=1, keepdims=True)

    iota_e = lax.broadcasted_iota(jnp.int32, (T, E), 1)
    a1 = jnp.argmax(probs, axis=1, keepdims=True).astype(jnp.int32)
    m1 = jnp.max(probs, axis=1, keepdims=True)
    eq1 = iota_e == a1
    probs2 = jnp.where(eq1, -1.0, probs)
    a2 = jnp.argmax(probs2, axis=1, keepdims=True).astype(jnp.int32)
    m2 = jnp.max(probs2, axis=1, keepdims=True)
    eq2 = iota_e == a2

    wsum = m1 + m2
    w1n_ref[...] = m1 / wsum
    w2n_ref[...] = m2 / wsum

    ohsum = (eq1 | eq2).astype(jnp.bfloat16)
    ti = lax.broadcasted_iota(jnp.int32, (T, T), 0)
    tj = lax.broadcasted_iota(jnp.int32, (T, T), 1)
    tril = (tj < ti).astype(jnp.bfloat16)
    cum = lax.dot_general(tril, ohsum, (((1,), (0,)), ((), ())),
                          preferred_element_type=jnp.float32)
    cum_i = cum.astype(jnp.int32)

    cnt = jnp.sum(eq1.astype(jnp.int32) + eq2.astype(jnp.int32), axis=0,
                  keepdims=True)
    pc = ((cnt + (MBLK - 1)) // MBLK) * MBLK
    lane = lax.broadcasted_iota(jnp.int32, (1, E), 1)
    off = jnp.zeros((1, E), jnp.int32)
    for k in range(1, E):
        off = off + jnp.where(lane >= k, jnp.roll(pc, k, axis=1), 0)

    rank1 = jnp.sum(jnp.where(eq1, cum_i, 0), axis=1, keepdims=True)
    rank2 = jnp.sum(jnp.where(eq2, cum_i, 0), axis=1, keepdims=True)
    off1 = jnp.sum(jnp.where(eq1, off, 0), axis=1, keepdims=True)
    off2 = jnp.sum(jnp.where(eq2, off, 0), axis=1, keepdims=True)
    pos1_ref[...] = off1 + rank1
    pos2_ref[...] = off2 + rank2

    bend = (off + pc) // MBLK
    iota_t = lax.broadcasted_iota(jnp.int32, (NTILES, E), 0)
    te = jnp.sum((iota_t >= bend).astype(jnp.int32), axis=1, keepdims=True)
    te_ref[...] = jnp.minimum(te, E - 1)
    nv_ref[...] = jnp.sum(pc, axis=1, keepdims=True) // MBLK


def _router(x, Wg):
    return pl.pallas_call(
        _router_body,
        out_shape=[
            jax.ShapeDtypeStruct((T, E), jnp.float32),
            jax.ShapeDtypeStruct((T, 1), jnp.int32),
            jax.ShapeDtypeStruct((T, 1), jnp.int32),
            jax.ShapeDtypeStruct((T, 1), jnp.float32),
            jax.ShapeDtypeStruct((T, 1), jnp.float32),
            jax.ShapeDtypeStruct((NTILES, 1), jnp.int32),
            jax.ShapeDtypeStruct((1, 1), jnp.int32),
        ],
    )(x, Wg)


def _sc_mesh():
    return plsc.VectorSubcoreMesh(core_axis_name="c", subcore_axis_name="s")


def _sc_dispatch(x2d, pos_cat):
    @functools.partial(
        pl.kernel, mesh=_sc_mesh(),
        out_type=jax.ShapeDtypeStruct((NPAD, D), jnp.float32),
        scratch_types=[pltpu.VMEM((_SC_CH,), jnp.int32),
                       pltpu.VMEM((_SC_CH, D), jnp.float32),
                       pltpu.SemaphoreType.DMA])
    def k(x_hbm, idx_hbm, out_hbm, idx_v, rows_v, sem):
        wid = lax.axis_index("s") * _SC_NC + lax.axis_index("c")
        base = wid * _SC_PER_W
        for c in range(_SC_PER_W // _SC_CH):
            b0 = base + c * _SC_CH
            tok = lax.rem(b0, T)
            pltpu.sync_copy(idx_hbm.at[pl.ds(b0, _SC_CH)], idx_v)
            pltpu.sync_copy(x_hbm.at[pl.ds(tok, _SC_CH)], rows_v)
            pltpu.async_copy(rows_v, out_hbm.at[idx_v], sem).wait()

    return k(x2d, pos_cat)


def _sc_combine(ys, pos_cat):
    @functools.partial(
        pl.kernel, mesh=_sc_mesh(),
        out_type=jax.ShapeDtypeStruct((NIDX, D), jnp.float32),
        scratch_types=[pltpu.VMEM((_SC_CH,), jnp.int32),
                       pltpu.VMEM((_SC_CH, D), jnp.float32),
                       pltpu.SemaphoreType.DMA])
    def k(ys_hbm, idx_hbm, out_hbm, idx_v, rows_v, sem):
        wid = lax.axis_index("s") * _SC_NC + lax.axis_index("c")
        base = wid * _SC_PER_W
        for c in range(_SC_PER_W // _SC_CH):
            b0 = base + c * _SC_CH
            pltpu.sync_copy(idx_hbm.at[pl.ds(b0, _SC_CH)], idx_v)
            pltpu.async_copy(ys_hbm.at[idx_v], rows_v, sem).wait()
            pltpu.sync_copy(rows_v, out_hbm.at[pl.ds(b0, _SC_CH)])

    return k(ys, pos_cat)


def _gelu_exact(x):
    return 0.5 * x * (1.0 + lax.erf(x * 0.7071067811865476))


NFSPLIT = 2
FCHUNK = FF // NFSPLIT


def _mlp_fused_body(te_ref, nv_ref, xs_ref, w1_ref, w2_ref, ys_ref):
    i = pl.program_id(0)
    f = pl.program_id(1)

    @pl.when(i < nv_ref[0])
    def _():
        xb = xs_ref[...].astype(jnp.bfloat16)
        w1 = w1_ref[0].astype(jnp.bfloat16)
        acc = lax.dot_general(xb, w1, (((1,), (1,)), ((), ())),
                              preferred_element_type=jnp.float32)
        hb = _gelu_exact(acc).astype(jnp.bfloat16)
        w2 = w2_ref[0].astype(jnp.bfloat16)
        part = lax.dot_general(hb, w2, (((1,), (1,)), ((), ())),
                               preferred_element_type=jnp.float32)

        @pl.when(f == 0)
        def _():
            ys_ref[...] = part

        @pl.when(f != 0)
        def _():
            ys_ref[...] += part


def _snake(i, f):
    return jnp.where(lax.rem(i, 2) == 0, f, NFSPLIT - 1 - f)


def _mlp_fused(xs, W1, W2, te, nv):
    grid_spec = pltpu.PrefetchScalarGridSpec(
        num_scalar_prefetch=2,
        grid=(NTILES, NFSPLIT),
        in_specs=[
            pl.BlockSpec((MBLK, D),
                         lambda i, f, te, nv: (jnp.minimum(i, nv[0]), 0)),
            pl.BlockSpec((1, FCHUNK, D),
                         lambda i, f, te, nv: (te[i], _snake(i, f), 0)),
            pl.BlockSpec((1, D, FCHUNK),
                         lambda i, f, te, nv: (te[i], 0, _snake(i, f))),
        ],
        out_specs=pl.BlockSpec(
            (MBLK, D), lambda i, f, te, nv: (jnp.minimum(i, nv[0]), 0)),
    )
    return pl.pallas_call(
        _mlp_fused_body,
        grid_spec=grid_spec,
        out_shape=jax.ShapeDtypeStruct((NPAD, D), jnp.float32),
        compiler_params=pltpu.CompilerParams(
            dimension_semantics=("parallel", "arbitrary")),
    )(te, nv, xs, W1, W2)


def _combine_body(g1_ref, g2_ref, w1_ref, w2_ref, o_ref):
    o_ref[...] = g1_ref[...] * w1_ref[...] + g2_ref[...] * w2_ref[...]


def _combine(g, w1n, w2n):
    nblk = T // MBLK
    return pl.pallas_call(
        _combine_body,
        grid=(nblk,),
        in_specs=[
            pl.BlockSpec((MBLK, D), lambda i: (i, 0)),
            pl.BlockSpec((MBLK, D), lambda i: (i + nblk, 0)),
            pl.BlockSpec((MBLK, 1), lambda i: (i, 0)),
            pl.BlockSpec((MBLK, 1), lambda i: (i, 0)),
        ],
        out_specs=pl.BlockSpec((MBLK, D), lambda i: (i, 0)),
        out_shape=jax.ShapeDtypeStruct((T, D), jnp.float32),
    )(g, g, w1n, w2n)


def kernel(hidden_states, Wg, W1, W2):
    b, s, d = hidden_states.shape
    x = hidden_states.reshape(T, D)

    logits, pos1, pos2, w1n, w2n, te, nv = _router(x, Wg)
    pos_cat = jnp.concatenate([pos1[:, 0], pos2[:, 0]])

    xs = _sc_dispatch(x, pos_cat)
    ys = _mlp_fused(xs, W1, W2, te[:, 0], nv[0])
    g = _sc_combine(ys, pos_cat)
    out = _combine(g, w1n, w2n)

    return (out.reshape(b, s, d), logits)
